# Initial kernel scaffold; baseline (speedup 1.0000x reference)
import jax
import jax.numpy as jnp
from jax.experimental import pallas as pl


def kernel(x, edge_index, edge_attr, W1, b1, W2, b2, Wc1, bc1, Wc2, bc2):
    raise NotImplementedError("write your pallas kernel here")


# speedup vs baseline: 9.0112x; 9.0112x over previous
import functools
import jax
import jax.numpy as jnp
from jax import lax
from jax.experimental import pallas as pl
from jax.experimental.pallas import tpu as pltpu
from jax.experimental.pallas import tpu_sc as plsc

N = 10000
E = 320000
D_IN = 128
H = 64
C = 3

NC, NS = 2, 16
NW = NC * NS
NPAD = 10240
EB = 128
EPT = 10112
EPAD = NW * EPT
NBLK = EPT // EB
RPT = NPAD // NS

_mesh = plsc.VectorSubcoreMesh(core_axis_name="c", subcore_axis_name="s")
_sc_params = pltpu.CompilerParams(use_tc_tiling_on_sc=False)


def _deg_body(dst_hbm, ew_hbm, out_hbm, idx_v, ewb_v, stripe_v, acc, sem):
    c = lax.axis_index("c")
    s = lax.axis_index("s")

    def _zero(i, _):
        stripe_v[pl.ds(i * 16, 16)] = jnp.zeros((16,), jnp.float32)
        return 0

    lax.fori_loop(0, RPT // 16, _zero, 0)
    pltpu.sync_copy(stripe_v, acc.at[pl.ds(s * RPT, RPT)])
    plsc.subcore_barrier()

    base = (c * NS + s) * EPT

    def _blk(b, _):
        off = base + b * EB
        pltpu.sync_copy(dst_hbm.at[pl.ds(off, EB)], idx_v.at[0])
        pltpu.sync_copy(ew_hbm.at[pl.ds(off, EB)], ewb_v.at[0])
        pltpu.sync_copy(ewb_v.at[0], acc.at[idx_v.at[0]], add=True)
        return 0

    lax.fori_loop(0, NBLK, _blk, 0)
    plsc.subcore_barrier()

    pltpu.sync_copy(acc.at[pl.ds(s * RPT, RPT)], stripe_v)
    pltpu.sync_copy(stripe_v, out_hbm.at[c, pl.ds(s * RPT, RPT)])


_deg_call = pl.kernel(
    _deg_body,
    out_type=jax.ShapeDtypeStruct((NC, NPAD), jnp.float32),
    mesh=_mesh,
    scratch_types=[
        pltpu.VMEM((1, EB), jnp.int32),
        pltpu.VMEM((1, EB), jnp.float32),
        pltpu.VMEM((RPT,), jnp.float32),
        pltpu.VMEM_SHARED((NPAD,), jnp.float32),
        pltpu.SemaphoreType.DMA,
    ],
    compiler_params=_sc_params,
)


def _agg_body(ys_hbm, src_hbm, dst_hbm, ew_hbm, out_hbm,
              sidx, didx, ewb, rows, acc, sem):
    c = lax.axis_index("c")
    s = lax.axis_index("s")

    def _zero(i, _):
        for j in range(4):
            rows[i, pl.ds(j * 16, 16)] = jnp.zeros((16,), jnp.float32)
        return 0

    lax.fori_loop(0, EB, _zero, 0)
    for k in range(RPT // EB):
        pltpu.sync_copy(rows, acc.at[pl.ds(s * RPT + k * EB, EB)])
    plsc.subcore_barrier()

    base = (c * NS + s) * EPT

    def _blk(b, _):
        off = base + b * EB
        pltpu.sync_copy(src_hbm.at[pl.ds(off, EB)], sidx.at[0])
        pltpu.sync_copy(dst_hbm.at[pl.ds(off, EB)], didx.at[0])
        pltpu.sync_copy(ew_hbm.at[pl.ds(off, EB)], ewb.at[0])
        pltpu.async_copy(ys_hbm.at[sidx.at[0]], rows, sem).wait()

        def _scale(g, _):
            wv = ewb[0, pl.ds(g * 16, 16)]
            for l in range(16):
                w = wv[l]
                r = g * 16 + l
                for j in range(4):
                    rows[r, pl.ds(j * 16, 16)] = rows[r, pl.ds(j * 16, 16)] * w
            return 0

        lax.fori_loop(0, EB // 16, _scale, 0)
        pltpu.sync_copy(rows, acc.at[didx.at[0]], add=True)
        return 0

    lax.fori_loop(0, NBLK, _blk, 0)
    plsc.subcore_barrier()

    for k in range(RPT // EB):
        r0 = s * RPT + k * EB
        pltpu.sync_copy(acc.at[pl.ds(r0, EB)], rows)
        pltpu.sync_copy(rows, out_hbm.at[c, pl.ds(r0, EB)])


_agg_call = pl.kernel(
    _agg_body,
    out_type=jax.ShapeDtypeStruct((NC, NPAD, H), jnp.float32),
    mesh=_mesh,
    scratch_types=[
        pltpu.VMEM((1, EB), jnp.int32),
        pltpu.VMEM((1, EB), jnp.int32),
        pltpu.VMEM((1, EB), jnp.float32),
        pltpu.VMEM((EB, H), jnp.float32),
        pltpu.VMEM_SHARED((NPAD, H), jnp.float32),
        pltpu.SemaphoreType.DMA,
    ],
    compiler_params=_sc_params,
)


_RB = 2048
_GRID = NPAD // _RB


def _mm1_body(x_ref, w_ref, dinv_ref, o_ref):
    o_ref[...] = jnp.dot(x_ref[...], w_ref[...],
                         preferred_element_type=jnp.float32) * dinv_ref[...]


def _mm2_body(p_ref, ys_ref, b_ref, w_ref, dinv_ref, o_ref):
    dinv = dinv_ref[...]
    h = jnp.maximum(
        dinv * (p_ref[0] + p_ref[1] + ys_ref[...]) + b_ref[...], 0.0)
    o_ref[...] = jnp.dot(h, w_ref[...],
                         preferred_element_type=jnp.float32) * dinv


def _mm3_body(p_ref, ys_ref, b_ref, wc1_ref, bc1_ref, wc2_ref, bc2_ref,
              dinv_ref, o_ref):
    dinv = dinv_ref[...]
    h2 = jnp.maximum(
        dinv * (p_ref[0] + p_ref[1] + ys_ref[...]) + b_ref[...], 0.0)
    t = jnp.maximum(
        jnp.dot(h2, wc1_ref[...], preferred_element_type=jnp.float32)
        + bc1_ref[...], 0.0)
    o_ref[...] = jnp.dot(t, wc2_ref[...],
                         preferred_element_type=jnp.float32) + bc2_ref[...]


_mm1 = pl.pallas_call(
    _mm1_body,
    grid=(_GRID,),
    in_specs=[
        pl.BlockSpec((_RB, D_IN), lambda i: (i, 0)),
        pl.BlockSpec((D_IN, H), lambda i: (0, 0)),
        pl.BlockSpec((_RB, 1), lambda i: (i, 0)),
    ],
    out_specs=pl.BlockSpec((_RB, H), lambda i: (i, 0)),
    out_shape=jax.ShapeDtypeStruct((N, H), jnp.float32),
)

_mm2 = pl.pallas_call(
    _mm2_body,
    grid=(_GRID,),
    in_specs=[
        pl.BlockSpec((NC, _RB, H), lambda i: (0, i, 0)),
        pl.BlockSpec((_RB, H), lambda i: (i, 0)),
        pl.BlockSpec((1, H), lambda i: (0, 0)),
        pl.BlockSpec((H, H), lambda i: (0, 0)),
        pl.BlockSpec((_RB, 1), lambda i: (i, 0)),
    ],
    out_specs=pl.BlockSpec((_RB, H), lambda i: (i, 0)),
    out_shape=jax.ShapeDtypeStruct((N, H), jnp.float32),
)

_mm3 = pl.pallas_call(
    _mm3_body,
    grid=(_GRID,),
    in_specs=[
        pl.BlockSpec((NC, _RB, H), lambda i: (0, i, 0)),
        pl.BlockSpec((_RB, H), lambda i: (i, 0)),
        pl.BlockSpec((1, H), lambda i: (0, 0)),
        pl.BlockSpec((H, H // 2), lambda i: (0, 0)),
        pl.BlockSpec((1, H // 2), lambda i: (0, 0)),
        pl.BlockSpec((H // 2, C), lambda i: (0, 0)),
        pl.BlockSpec((1, C), lambda i: (0, 0)),
        pl.BlockSpec((_RB, 1), lambda i: (i, 0)),
    ],
    out_specs=pl.BlockSpec((_RB, C), lambda i: (i, 0)),
    out_shape=jax.ShapeDtypeStruct((N, C), jnp.float32),
)


@jax.jit
def kernel(x, edge_index, edge_attr, W1, b1, W2, b2, Wc1, bc1, Wc2, bc2):
    src = edge_index[0]
    dst = edge_index[1]
    ew = jnp.squeeze(edge_attr, axis=-1)

    pad = EPAD - E
    srcp = jnp.concatenate([src, jnp.zeros((pad,), src.dtype)])
    dstp = jnp.concatenate([dst, jnp.zeros((pad,), dst.dtype)])
    ewp = jnp.concatenate([ew, jnp.zeros((pad,), ew.dtype)])

    deg_parts = _deg_call(dstp, ewp)
    deg = deg_parts[0] + deg_parts[1] + 1.0
    dinv = lax.rsqrt(deg).reshape(NPAD, 1)

    ys1 = _mm1(x, W1, dinv)
    p1 = _agg_call(ys1, srcp, dstp, ewp)
    ys2 = _mm2(p1, ys1, b1.reshape(1, H), W2, dinv)
    p2 = _agg_call(ys2, srcp, dstp, ewp)
    out = _mm3(p2, ys2, b2.reshape(1, H), Wc1,
               bc1.reshape(1, H // 2), Wc2, bc2.reshape(1, C), dinv)
    return out


# speedup vs baseline: 13.0878x; 1.4524x over previous
import functools
import jax
import jax.numpy as jnp
from jax import lax
from jax.experimental import pallas as pl
from jax.experimental.pallas import tpu as pltpu
from jax.experimental.pallas import tpu_sc as plsc

N = 10000
E = 320000
D_IN = 128
H = 64
C = 3

NC, NS = 2, 16
NW = NC * NS
NPAD = 10240
EB = 128
NBLK = 80
EPT = NBLK * EB
EPAD = NW * EPT
RPT = NPAD // NS

_mesh = plsc.VectorSubcoreMesh(core_axis_name="c", subcore_axis_name="s")
_sc_params = pltpu.CompilerParams(use_tc_tiling_on_sc=False)


def _deg_body(dst_hbm, ew_hbm, out_hbm, didx, ewb, stripe_v, acc, sem, lsem):
    c = lax.axis_index("c")
    s = lax.axis_index("s")
    w = c * NS + s

    def _zero(i, _):
        stripe_v[pl.ds(i * 16, 16)] = jnp.zeros((16,), jnp.float32)
        return 0

    lax.fori_loop(0, RPT // 16, _zero, 0)
    pltpu.sync_copy(stripe_v, acc.at[pl.ds(s * RPT, RPT)])
    pltpu.async_copy(dst_hbm.at[w], didx, lsem)
    pltpu.async_copy(ew_hbm.at[w], ewb, lsem)
    pltpu.make_async_copy(dst_hbm.at[w], didx, lsem).wait()
    pltpu.make_async_copy(ew_hbm.at[w], ewb, lsem).wait()
    plsc.subcore_barrier()

    K = 8

    def _grp(g, _):
        b0 = g * K
        cps = [
            pltpu.async_copy(ewb.at[b0 + j], acc.at[didx.at[b0 + j]],
                             sem, add=True)
            for j in range(K)
        ]
        for cp in cps:
            cp.wait()
        return 0

    lax.fori_loop(0, NBLK // K, _grp, 0)
    plsc.subcore_barrier()

    pltpu.sync_copy(acc.at[pl.ds(s * RPT, RPT)], stripe_v)
    pltpu.sync_copy(stripe_v, out_hbm.at[c, pl.ds(s * RPT, RPT)])


_deg_call = pl.kernel(
    _deg_body,
    out_type=jax.ShapeDtypeStruct((NC, NPAD), jnp.float32),
    mesh=_mesh,
    scratch_types=[
        pltpu.VMEM((NBLK, EB), jnp.int32),
        pltpu.VMEM((NBLK, EB), jnp.float32),
        pltpu.VMEM((RPT,), jnp.float32),
        pltpu.VMEM_SHARED((NPAD,), jnp.float32),
        pltpu.SemaphoreType.DMA,
        pltpu.SemaphoreType.DMA,
    ],
    compiler_params=_sc_params,
)


def _agg_body(ys_hbm, src_hbm, dst_hbm, ew_hbm, out_hbm,
              sidx, didx, ewb, rows0, rows1, acc, sem0, sem1, lsem):
    c = lax.axis_index("c")
    s = lax.axis_index("s")
    w = c * NS + s

    def _zero(i, _):
        for j in range(4):
            rows0[i, pl.ds(j * 16, 16)] = jnp.zeros((16,), jnp.float32)
        return 0

    lax.fori_loop(0, EB, _zero, 0)
    pltpu.async_copy(src_hbm.at[w], sidx.at[pl.ds(0, NBLK)], lsem)
    pltpu.async_copy(dst_hbm.at[w], didx, lsem)
    pltpu.async_copy(ew_hbm.at[w], ewb, lsem)
    for j in range(EB // 16):
        sidx[NBLK, pl.ds(j * 16, 16)] = jnp.zeros((16,), jnp.int32)
    for k in range(RPT // EB):
        pltpu.sync_copy(rows0, acc.at[pl.ds(s * RPT + k * EB, EB)])
    pltpu.make_async_copy(src_hbm.at[w], sidx.at[pl.ds(0, NBLK)], lsem).wait()
    pltpu.make_async_copy(dst_hbm.at[w], didx, lsem).wait()
    pltpu.make_async_copy(ew_hbm.at[w], ewb, lsem).wait()
    plsc.subcore_barrier()

    bufs = (rows0, rows1)
    sems = (sem0, sem1)

    def _gather(b, buf, sem):
        pltpu.async_copy(ys_hbm.at[sidx.at[b]], buf, sem)

    def _wait(b, buf, sem):
        pltpu.make_async_copy(ys_hbm.at[sidx.at[b]], buf, sem).wait()

    def _scale(b, buf):
        def _grp(g, _):
            wv = ewb[b, pl.ds(g * 16, 16)]
            for l in range(16):
                sw = wv[l]
                r = g * 16 + l
                for j in range(4):
                    buf[r, pl.ds(j * 16, 16)] = buf[r, pl.ds(j * 16, 16)] * sw
            return 0

        lax.fori_loop(0, EB // 16, _grp, 0)

    _gather(0, rows0, sem0)

    def _pair(i, _):
        b = 2 * i
        for k in range(2):
            bk = b + k
            _gather(bk + 1, bufs[(k + 1) % 2], sems[(k + 1) % 2])
            _wait(bk, bufs[k], sems[k])
            _scale(bk, bufs[k])
            pltpu.sync_copy(bufs[k], acc.at[didx.at[bk]], add=True)
        return 0

    lax.fori_loop(0, NBLK // 2, _pair, 0)
    _wait(NBLK, rows0, sem0)
    plsc.subcore_barrier()

    for k in range(RPT // EB):
        r0 = s * RPT + k * EB
        pltpu.sync_copy(acc.at[pl.ds(r0, EB)], rows0)
        pltpu.sync_copy(rows0, out_hbm.at[c, pl.ds(r0, EB)])


_agg_call = pl.kernel(
    _agg_body,
    out_type=jax.ShapeDtypeStruct((NC, NPAD, H), jnp.float32),
    mesh=_mesh,
    scratch_types=[
        pltpu.VMEM((NBLK + 1, EB), jnp.int32),
        pltpu.VMEM((NBLK, EB), jnp.int32),
        pltpu.VMEM((NBLK, EB), jnp.float32),
        pltpu.VMEM((EB, H), jnp.float32),
        pltpu.VMEM((EB, H), jnp.float32),
        pltpu.VMEM_SHARED((NPAD, H), jnp.float32),
        pltpu.SemaphoreType.DMA,
        pltpu.SemaphoreType.DMA,
        pltpu.SemaphoreType.DMA,
    ],
    compiler_params=_sc_params,
)


_RB = 2048
_GRID = NPAD // _RB


def _mm1_body(x_ref, w_ref, dinv_ref, o_ref):
    o_ref[...] = jnp.dot(x_ref[...], w_ref[...],
                         preferred_element_type=jnp.float32) * dinv_ref[...]


def _mm2_body(p_ref, ys_ref, b_ref, w_ref, dinv_ref, o_ref):
    dinv = dinv_ref[...]
    h = jnp.maximum(
        dinv * (p_ref[0] + p_ref[1] + ys_ref[...]) + b_ref[...], 0.0)
    o_ref[...] = jnp.dot(h, w_ref[...],
                         preferred_element_type=jnp.float32) * dinv


def _mm3_body(p_ref, ys_ref, b_ref, wc1_ref, bc1_ref, wc2_ref, bc2_ref,
              dinv_ref, o_ref):
    dinv = dinv_ref[...]
    h2 = jnp.maximum(
        dinv * (p_ref[0] + p_ref[1] + ys_ref[...]) + b_ref[...], 0.0)
    t = jnp.maximum(
        jnp.dot(h2, wc1_ref[...], preferred_element_type=jnp.float32)
        + bc1_ref[...], 0.0)
    o_ref[...] = jnp.dot(t, wc2_ref[...],
                         preferred_element_type=jnp.float32) + bc2_ref[...]


_mm1 = pl.pallas_call(
    _mm1_body,
    grid=(_GRID,),
    in_specs=[
        pl.BlockSpec((_RB, D_IN), lambda i: (i, 0)),
        pl.BlockSpec((D_IN, H), lambda i: (0, 0)),
        pl.BlockSpec((_RB, 1), lambda i: (i, 0)),
    ],
    out_specs=pl.BlockSpec((_RB, H), lambda i: (i, 0)),
    out_shape=jax.ShapeDtypeStruct((N, H), jnp.float32),
)

_mm2 = pl.pallas_call(
    _mm2_body,
    grid=(_GRID,),
    in_specs=[
        pl.BlockSpec((NC, _RB, H), lambda i: (0, i, 0)),
        pl.BlockSpec((_RB, H), lambda i: (i, 0)),
        pl.BlockSpec((1, H), lambda i: (0, 0)),
        pl.BlockSpec((H, H), lambda i: (0, 0)),
        pl.BlockSpec((_RB, 1), lambda i: (i, 0)),
    ],
    out_specs=pl.BlockSpec((_RB, H), lambda i: (i, 0)),
    out_shape=jax.ShapeDtypeStruct((N, H), jnp.float32),
)

_mm3 = pl.pallas_call(
    _mm3_body,
    grid=(_GRID,),
    in_specs=[
        pl.BlockSpec((NC, _RB, H), lambda i: (0, i, 0)),
        pl.BlockSpec((_RB, H), lambda i: (i, 0)),
        pl.BlockSpec((1, H), lambda i: (0, 0)),
        pl.BlockSpec((H, H // 2), lambda i: (0, 0)),
        pl.BlockSpec((1, H // 2), lambda i: (0, 0)),
        pl.BlockSpec((H // 2, C), lambda i: (0, 0)),
        pl.BlockSpec((1, C), lambda i: (0, 0)),
        pl.BlockSpec((_RB, 1), lambda i: (i, 0)),
    ],
    out_specs=pl.BlockSpec((_RB, C), lambda i: (i, 0)),
    out_shape=jax.ShapeDtypeStruct((N, C), jnp.float32),
)


@jax.jit
def kernel(x, edge_index, edge_attr, W1, b1, W2, b2, Wc1, bc1, Wc2, bc2):
    src = edge_index[0]
    dst = edge_index[1]
    ew = jnp.squeeze(edge_attr, axis=-1)

    pad = EPAD - E
    srcp = jnp.concatenate(
        [src, jnp.zeros((pad,), src.dtype)]).reshape(NW, NBLK, EB)
    dstp = jnp.concatenate(
        [dst, jnp.zeros((pad,), dst.dtype)]).reshape(NW, NBLK, EB)
    ewp = jnp.concatenate(
        [ew, jnp.zeros((pad,), ew.dtype)]).reshape(NW, NBLK, EB)

    deg_parts = _deg_call(dstp, ewp)
    deg = deg_parts[0] + deg_parts[1] + 1.0
    dinv = lax.rsqrt(deg).reshape(NPAD, 1)

    ys1 = _mm1(x, W1, dinv)
    p1 = _agg_call(ys1, srcp, dstp, ewp)
    ys2 = _mm2(p1, ys1, b1.reshape(1, H), W2, dinv)
    p2 = _agg_call(ys2, srcp, dstp, ewp)
    out = _mm3(p2, ys2, b2.reshape(1, H), Wc1,
               bc1.reshape(1, H // 2), Wc2, bc2.reshape(1, C), dinv)
    return out
